# baseline (speedup 1.0000x reference)
import functools
import math

import jax
import jax.numpy as jnp
from jax import lax
from jax.experimental import pallas as pl
from jax.experimental.pallas import tpu as pltpu
from jax.experimental.pallas import tpu_sc as plsc

BATCH = 4096
SEQ = 200
DIM = 64
LANES = 16
NUM_CORES = 2
NUM_SUBCORES = 16
NW = NUM_CORES * NUM_SUBCORES
ROWS_PER_W = BATCH * SEQ // NW
CHUNK = 128
NCHUNK = ROWS_PER_W // CHUNK
VPR = DIM // LANES
SCALE = math.sqrt(DIM)


def _body(tok_hbm, lex_hbm, pos_hbm, out_hbm,
          idx_v, pos_v, buf0, buf1, gsem0, gsem1):
    wid = lax.axis_index("s") * NUM_CORES + lax.axis_index("c")
    base = wid * ROWS_PER_W

    pltpu.sync_copy(tok_hbm.at[pl.ds(base, ROWS_PER_W)], idx_v)
    pltpu.sync_copy(pos_hbm.at[pl.ds(0, SEQ)], pos_v)

    bufs = (buf0, buf1)
    sems = (gsem0, gsem1)

    def fire(g, buf, sem):
        pltpu.make_async_copy(
            lex_hbm.at[idx_v.at[pl.ds(g * CHUNK, CHUNK)]], buf, sem).start()

    def wait(buf, sem):
        pltpu.make_async_copy(
            lex_hbm.at[idx_v.at[pl.ds(0, CHUNK)]], buf, sem).wait()

    def compute(g, buf):
        def row(i, _):
            s = lax.rem(g * CHUNK + i, SEQ)
            for k in range(VPR):
                sl = pl.ds(k * LANES, LANES)
                buf[i, sl] = buf[i, sl] * SCALE + pos_v[s, sl]
            return _
        lax.fori_loop(0, CHUNK, row, None)

    fire(0, bufs[0], sems[0])

    def pair(j, _):
        for b in range(2):
            g = 2 * j + b
            nxt = g + 1
            if b == 0:
                @pl.when(nxt < NCHUNK)
                def _():
                    fire(nxt, bufs[1], sems[1])
            else:
                @pl.when(nxt < NCHUNK)
                def _():
                    fire(nxt, bufs[0], sems[0])
            wait(bufs[b], sems[b])
            compute(g, bufs[b])
            pltpu.sync_copy(bufs[b], out_hbm.at[pl.ds(base + g * CHUNK, CHUNK)])
        return _

    lax.fori_loop(0, NCHUNK // 2, pair, None)


@jax.jit
def _sc_embed(tok_flat, lex, pos):
    mesh = plsc.VectorSubcoreMesh(core_axis_name="c", subcore_axis_name="s")
    kern = functools.partial(
        pl.kernel,
        out_type=jax.ShapeDtypeStruct((BATCH * SEQ, DIM), jnp.float32),
        mesh=mesh,
        compiler_params=pltpu.CompilerParams(use_tc_tiling_on_sc=False),
        scratch_types=[
            pltpu.VMEM((ROWS_PER_W,), jnp.int32),
            pltpu.VMEM((SEQ, DIM), jnp.float32),
            pltpu.VMEM((CHUNK, DIM), jnp.float32),
            pltpu.VMEM((CHUNK, DIM), jnp.float32),
            pltpu.SemaphoreType.DMA,
            pltpu.SemaphoreType.DMA,
        ],
    )(_body)
    return kern(tok_flat, lex, pos)


def kernel(token_indices, lexical_weight, positional_weight):
    b, s = token_indices.shape
    tok_flat = token_indices.reshape(b * s).astype(jnp.int32)
    out = _sc_embed(tok_flat, lexical_weight, positional_weight)
    return out.reshape(b, s, DIM)


# speedup vs baseline: 1.2458x; 1.2458x over previous
import functools
import math

import jax
import jax.numpy as jnp
from jax import lax
from jax.experimental import pallas as pl
from jax.experimental.pallas import tpu as pltpu
from jax.experimental.pallas import tpu_sc as plsc

BATCH = 4096
SEQ = 200
DIM = 64
LANES = 16
NUM_CORES = 2
NUM_SUBCORES = 16
NW = NUM_CORES * NUM_SUBCORES
ROWS_PER_W = BATCH * SEQ // NW
CHUNK = 128
NCHUNK = ROWS_PER_W // CHUNK
VPR = DIM // LANES
SCALE = math.sqrt(DIM)


def _body(tok_hbm, lex_hbm, pos_hbm, out_hbm,
          idx_v, pos_v, buf0, buf1, gsem0, gsem1, osem0, osem1):
    wid = lax.axis_index("s") * NUM_CORES + lax.axis_index("c")
    base = wid * ROWS_PER_W

    pltpu.sync_copy(tok_hbm.at[pl.ds(base, ROWS_PER_W)], idx_v)
    pltpu.sync_copy(pos_hbm.at[pl.ds(0, SEQ)], pos_v)

    bufs = (buf0, buf1)
    gsems = (gsem0, gsem1)
    osems = (osem0, osem1)

    def fire(g, b):
        pltpu.make_async_copy(
            lex_hbm.at[idx_v.at[pl.ds(g * CHUNK, CHUNK)]], bufs[b],
            gsems[b]).start()

    def wait_gather(b):
        pltpu.make_async_copy(
            lex_hbm.at[idx_v.at[pl.ds(0, CHUNK)]], bufs[b], gsems[b]).wait()

    def start_out(g, b):
        pltpu.make_async_copy(
            bufs[b], out_hbm.at[pl.ds(base + g * CHUNK, CHUNK)],
            osems[b]).start()

    def wait_out(b):
        pltpu.make_async_copy(
            bufs[b], out_hbm.at[pl.ds(base, CHUNK)], osems[b]).wait()

    def compute(g, b):
        buf = bufs[b]

        @plsc.parallel_loop(0, CHUNK, unroll=4)
        def row(i):
            s = lax.rem(g * CHUNK + i, SEQ)
            for k in range(VPR):
                sl = pl.ds(k * LANES, LANES)
                buf[i, sl] = buf[i, sl] * SCALE + pos_v[s, sl]

    fire(0, 0)

    def pair(j, _):
        for b in range(2):
            g = 2 * j + b
            nxt = g + 1
            if b == 0:
                @pl.when((nxt < NCHUNK) & (j >= 1))
                def _():
                    wait_out(1)
            else:
                @pl.when(nxt < NCHUNK)
                def _():
                    wait_out(0)

            @pl.when(nxt < NCHUNK)
            def _():
                fire(nxt, 1 - b)

            wait_gather(b)
            compute(g, b)
            start_out(g, b)
        return _

    lax.fori_loop(0, NCHUNK // 2, pair, None)
    wait_out(0)
    wait_out(1)


@jax.jit
def _sc_embed(tok_flat, lex, pos):
    mesh = plsc.VectorSubcoreMesh(core_axis_name="c", subcore_axis_name="s")
    kern = functools.partial(
        pl.kernel,
        out_type=jax.ShapeDtypeStruct((BATCH * SEQ, DIM), jnp.float32),
        mesh=mesh,
        compiler_params=pltpu.CompilerParams(use_tc_tiling_on_sc=False),
        scratch_types=[
            pltpu.VMEM((ROWS_PER_W,), jnp.int32),
            pltpu.VMEM((SEQ, DIM), jnp.float32),
            pltpu.VMEM((CHUNK, DIM), jnp.float32),
            pltpu.VMEM((CHUNK, DIM), jnp.float32),
            pltpu.SemaphoreType.DMA,
            pltpu.SemaphoreType.DMA,
            pltpu.SemaphoreType.DMA,
            pltpu.SemaphoreType.DMA,
        ],
    )(_body)
    return kern(tok_flat, lex, pos)


def kernel(token_indices, lexical_weight, positional_weight):
    b, s = token_indices.shape
    tok_flat = token_indices.reshape(b * s).astype(jnp.int32)
    out = _sc_embed(tok_flat, lexical_weight, positional_weight)
    return out.reshape(b, s, DIM)


# speedup vs baseline: 1.2718x; 1.0209x over previous
import functools
import math

import jax
import jax.numpy as jnp
from jax import lax
from jax.experimental import pallas as pl
from jax.experimental.pallas import tpu as pltpu
from jax.experimental.pallas import tpu_sc as plsc

BATCH = 4096
SEQ = 200
DIM = 64
LANES = 16
NUM_CORES = 2
NUM_SUBCORES = 16
NW = NUM_CORES * NUM_SUBCORES
SEQ_PER_W = BATCH // NW
SPLIT = 104
VPR = DIM // LANES
SCALE = math.sqrt(DIM)


def _body(tok_hbm, lex_hbm, pos_hbm, out_hbm,
          idx_v, pos_v, buf0, buf1,
          gsem0a, gsem0b, gsem1a, gsem1b, osem0, osem1):
    wid = lax.axis_index("s") * NUM_CORES + lax.axis_index("c")
    seq0 = wid * SEQ_PER_W

    pltpu.sync_copy(tok_hbm.at[pl.ds(seq0, SEQ_PER_W)], idx_v)
    pltpu.sync_copy(pos_hbm.at[pl.ds(0, SEQ)], pos_v)

    bufs = (buf0, buf1)
    gsems = ((gsem0a, gsem0b), (gsem1a, gsem1b))
    osems = (osem0, osem1)
    halves = ((0, SPLIT), (SPLIT, SEQ - SPLIT))

    def fire(i, b):
        for h, (lo, n) in enumerate(halves):
            pltpu.make_async_copy(
                lex_hbm.at[idx_v.at[i, pl.ds(lo, n)]],
                bufs[b].at[pl.ds(lo, n)], gsems[b][h]).start()

    def wait_gather(b, h):
        lo, n = halves[h]
        pltpu.make_async_copy(
            lex_hbm.at[idx_v.at[0, pl.ds(lo, n)]],
            bufs[b].at[pl.ds(lo, n)], gsems[b][h]).wait()

    def start_out(i, b):
        pltpu.make_async_copy(bufs[b], out_hbm.at[seq0 + i], osems[b]).start()

    def wait_out(b):
        pltpu.make_async_copy(bufs[b], out_hbm.at[seq0], osems[b]).wait()

    def compute(b, h):
        buf = bufs[b]
        lo, n = halves[h]

        @plsc.parallel_loop(lo, lo + n, unroll=8)
        def row(i):
            for k in range(VPR):
                sl = pl.ds(k * LANES, LANES)
                buf[i, sl] = buf[i, sl] * SCALE + pos_v[i, sl]

    fire(0, 0)

    def pair(j, _):
        for b in range(2):
            i = 2 * j + b
            nxt = i + 1
            if b == 0:
                @pl.when((nxt < SEQ_PER_W) & (j >= 1))
                def _():
                    wait_out(1)
            else:
                @pl.when(nxt < SEQ_PER_W)
                def _():
                    wait_out(0)

            @pl.when(nxt < SEQ_PER_W)
            def _():
                fire(nxt, 1 - b)

            wait_gather(b, 0)
            compute(b, 0)
            wait_gather(b, 1)
            compute(b, 1)
            start_out(i, b)
        return _

    lax.fori_loop(0, SEQ_PER_W // 2, pair, None)
    wait_out(0)
    wait_out(1)


@jax.jit
def _sc_embed(tok, lex, pos):
    mesh = plsc.VectorSubcoreMesh(core_axis_name="c", subcore_axis_name="s")
    kern = functools.partial(
        pl.kernel,
        out_type=jax.ShapeDtypeStruct((BATCH, SEQ, DIM), jnp.float32),
        mesh=mesh,
        compiler_params=pltpu.CompilerParams(use_tc_tiling_on_sc=False),
        scratch_types=[
            pltpu.VMEM((SEQ_PER_W, SEQ), jnp.int32),
            pltpu.VMEM((SEQ, DIM), jnp.float32),
            pltpu.VMEM((SEQ, DIM), jnp.float32),
            pltpu.VMEM((SEQ, DIM), jnp.float32),
            pltpu.SemaphoreType.DMA,
            pltpu.SemaphoreType.DMA,
            pltpu.SemaphoreType.DMA,
            pltpu.SemaphoreType.DMA,
            pltpu.SemaphoreType.DMA,
            pltpu.SemaphoreType.DMA,
        ],
    )(_body)
    return kern(tok, lex, pos)


def kernel(token_indices, lexical_weight, positional_weight):
    return _sc_embed(token_indices, lexical_weight, positional_weight)
